# Initial kernel scaffold; baseline (speedup 1.0000x reference)
import jax
import jax.numpy as jnp
from jax.experimental import pallas as pl


def kernel(node_repr, head, rel, tail, sector, entity_type_id, rel_emb_W, sector_emb_W, type_emb_W):
    raise NotImplementedError("write your pallas kernel here")


# speedup vs baseline: 10.1352x; 10.1352x over previous
import functools

import jax
import jax.numpy as jnp
from jax import lax
from jax.experimental import pallas as pl
from jax.experimental.pallas import tpu as pltpu
from jax.experimental.pallas import tpu_sc as plsc

_N_NODES = 10000
_N_EDGES = 320000
_HIDDEN = 128

_NC = 2
_NS = 16
_NW = _NC * _NS
_L = 16

_EPW = _N_EDGES // _NW
_C = 80
_NCHUNK = _EPW // _C


def _augment_body(node_ref, etype_ref, typew_ref, out_ref):
    et = etype_ref[...]
    k = lax.broadcasted_iota(jnp.int32, (et.shape[0], typew_ref.shape[0]), 1)
    onehot = (et == k).astype(jnp.float32)
    out_ref[...] = node_ref[...] + jnp.dot(
        onehot, typew_ref[...], preferred_element_type=jnp.float32)


def _edge_body(a_hbm, head_hbm, tail_hbm, rel_hbm, sec_hbm, relw_hbm, secw_hbm,
               out_hbm, head_v, tail_v, rel_v, sec_v, hbuf, tbuf, rbuf, sbuf,
               partial, out_v, sem):
    wid = lax.axis_index("s") * _NC + lax.axis_index("c")
    base = wid * _EPW
    pltpu.sync_copy(head_hbm.at[pl.ds(base, _EPW)], head_v)
    pltpu.sync_copy(tail_hbm.at[pl.ds(base, _EPW)], tail_v)
    pltpu.sync_copy(rel_hbm.at[pl.ds(base, _EPW)], rel_v)
    pltpu.sync_copy(sec_hbm.at[pl.ds(base, _EPW)], sec_v)

    def chunk_body(c, carry):
        off = c * _C
        pltpu.async_copy(a_hbm.at[head_v.at[pl.ds(off, _C)]], hbuf, sem).wait()
        pltpu.async_copy(a_hbm.at[tail_v.at[pl.ds(off, _C)]], tbuf, sem).wait()
        pltpu.async_copy(relw_hbm.at[rel_v.at[pl.ds(off, _C)]], rbuf, sem).wait()
        pltpu.async_copy(secw_hbm.at[sec_v.at[pl.ds(off, _C)]], sbuf, sem).wait()

        def blk_body(j, carry2):
            def lane_body(l, carry3):
                e = j * _L + l
                acc = jnp.zeros((_L,), jnp.float32)
                for g in range(_HIDDEN // _L):
                    h = hbuf[e, pl.ds(g * _L, _L)]
                    r = rbuf[e, pl.ds(g * _L, _L)] + sbuf[e, pl.ds(g * _L, _L)]
                    t = tbuf[e, pl.ds(g * _L, _L)]
                    acc = acc + h * r * t
                partial[pl.ds(l * _L, _L)] = acc
                return carry3

            lax.fori_loop(0, _L, lane_body, 0)
            rowbase = lax.iota(jnp.int32, _L) * _L
            score = jnp.zeros((_L,), jnp.float32)
            for c in range(_L):
                score = score + plsc.load_gather(partial, [rowbase + c])
            out_v[pl.ds(off + j * _L, _L)] = score
            return carry2

        lax.fori_loop(0, _C // _L, blk_body, 0)
        return carry

    lax.fori_loop(0, _NCHUNK, chunk_body, 0)
    pltpu.sync_copy(out_v, out_hbm.at[pl.ds(base, _EPW)])


_edge_kernel = functools.partial(
    pl.kernel,
    out_type=jax.ShapeDtypeStruct((_N_EDGES,), jnp.float32),
    mesh=plsc.VectorSubcoreMesh(
        core_axis_name="c", subcore_axis_name="s",
        num_cores=_NC, num_subcores=_NS),
    compiler_params=pltpu.CompilerParams(needs_layout_passes=False),
    scratch_types=[
        pltpu.VMEM((_EPW,), jnp.int32),
        pltpu.VMEM((_EPW,), jnp.int32),
        pltpu.VMEM((_EPW,), jnp.int32),
        pltpu.VMEM((_EPW,), jnp.int32),
        pltpu.VMEM((_C, _HIDDEN), jnp.float32),
        pltpu.VMEM((_C, _HIDDEN), jnp.float32),
        pltpu.VMEM((_C, _HIDDEN), jnp.float32),
        pltpu.VMEM((_C, _HIDDEN), jnp.float32),
        pltpu.VMEM((_L * _L,), jnp.float32),
        pltpu.VMEM((_EPW,), jnp.float32),
        pltpu.SemaphoreType.DMA,
    ],
)(_edge_body)


def kernel(node_repr, head, rel, tail, sector, entity_type_id,
           rel_emb_W, sector_emb_W, type_emb_W):
    etype2d = entity_type_id.astype(jnp.int32).reshape(_N_NODES, 1)
    aug = pl.pallas_call(
        _augment_body,
        out_shape=jax.ShapeDtypeStruct((_N_NODES, _HIDDEN), jnp.float32),
    )(node_repr, etype2d, type_emb_W)
    return _edge_kernel(
        aug,
        head.astype(jnp.int32), tail.astype(jnp.int32),
        rel.astype(jnp.int32), sector.astype(jnp.int32),
        rel_emb_W, sector_emb_W)


# speedup vs baseline: 30.6912x; 3.0282x over previous
import functools

import jax
import jax.numpy as jnp
from jax import lax
from jax.experimental import pallas as pl
from jax.experimental.pallas import tpu as pltpu
from jax.experimental.pallas import tpu_sc as plsc

_N_NODES = 10000
_N_EDGES = 320000
_HIDDEN = 128
_N_SEC = 50

_NC = 2
_NS = 16
_NW = _NC * _NS
_L = 16

_EPW = _N_EDGES // _NW
_C = 80
_NCHUNK = _EPW // _C


def _augment_body(node_ref, etype_ref, typew_ref, out_ref):
    et = etype_ref[...]
    k = lax.broadcasted_iota(jnp.int32, (et.shape[0], typew_ref.shape[0]), 1)
    onehot = (et == k).astype(jnp.float32)
    out_ref[...] = node_ref[...] + jnp.dot(
        onehot, typew_ref[...], preferred_element_type=jnp.float32)


def _relsec_body(relw_ref, secw_ref, out_ref):
    out_ref[...] = relw_ref[...][:, None, :] + secw_ref[...][None, :, :]


def _edge_body(a_hbm, head_hbm, tail_hbm, rel_hbm, sec_hbm, r_hbm,
               out_hbm, head_v, tail_v, rs_v, sec_v, hbuf, tbuf, rbuf,
               partial, out_v, sem):
    wid = lax.axis_index("s") * _NC + lax.axis_index("c")
    base = wid * _EPW
    pltpu.sync_copy(head_hbm.at[pl.ds(base, _EPW)], head_v)
    pltpu.sync_copy(tail_hbm.at[pl.ds(base, _EPW)], tail_v)
    pltpu.sync_copy(rel_hbm.at[pl.ds(base, _EPW)], rs_v)
    pltpu.sync_copy(sec_hbm.at[pl.ds(base, _EPW)], sec_v)

    def rs_body(k, carry):
        sl = pl.ds(k * _L, _L)
        rs_v[sl] = rs_v[sl] * _N_SEC + sec_v[sl]
        return carry

    lax.fori_loop(0, _EPW // _L, rs_body, 0)

    def issue(c, slot):
        off = c * _C
        cps = (
            pltpu.async_copy(a_hbm.at[head_v.at[pl.ds(off, _C)]],
                             hbuf.at[slot], sem),
            pltpu.async_copy(a_hbm.at[tail_v.at[pl.ds(off, _C)]],
                             tbuf.at[slot], sem),
            pltpu.async_copy(r_hbm.at[rs_v.at[pl.ds(off, _C)]],
                             rbuf.at[slot], sem),
        )
        return cps

    def compute(c, slot):
        off = c * _C

        def blk_body(j, carry2):
            def lane_body(l, carry3):
                e = j * _L + l
                acc = jnp.zeros((_L,), jnp.float32)
                for g in range(_HIDDEN // _L):
                    sl = pl.ds(g * _L, _L)
                    h = hbuf[slot, e, sl]
                    r = rbuf[slot, e, sl]
                    t = tbuf[slot, e, sl]
                    acc = acc + h * r * t
                partial[pl.ds(l * _L, _L)] = acc
                return carry3

            lax.fori_loop(0, _L, lane_body, 0)
            rowbase = lax.iota(jnp.int32, _L) * _L
            score = jnp.zeros((_L,), jnp.float32)
            for cc in range(_L):
                score = score + plsc.load_gather(partial, [rowbase + cc])
            out_v[pl.ds(off + j * _L, _L)] = score
            return carry2

        lax.fori_loop(0, _C // _L, blk_body, 0)

    def chunk_body(c, carry):
        slot = lax.rem(c, 2)
        cps = issue(c, slot)

        @pl.when(c > 0)
        def _():
            compute(c - 1, 1 - slot)

        for cp in cps:
            cp.wait()
        return carry

    lax.fori_loop(0, _NCHUNK, chunk_body, 0)
    compute(_NCHUNK - 1, lax.rem(_NCHUNK - 1, 2))
    pltpu.sync_copy(out_v, out_hbm.at[pl.ds(base, _EPW)])


_edge_kernel = functools.partial(
    pl.kernel,
    out_type=jax.ShapeDtypeStruct((_N_EDGES,), jnp.float32),
    mesh=plsc.VectorSubcoreMesh(
        core_axis_name="c", subcore_axis_name="s",
        num_cores=_NC, num_subcores=_NS),
    compiler_params=pltpu.CompilerParams(needs_layout_passes=False),
    scratch_types=[
        pltpu.VMEM((_EPW,), jnp.int32),
        pltpu.VMEM((_EPW,), jnp.int32),
        pltpu.VMEM((_EPW,), jnp.int32),
        pltpu.VMEM((_EPW,), jnp.int32),
        pltpu.VMEM((2, _C, _HIDDEN), jnp.float32),
        pltpu.VMEM((2, _C, _HIDDEN), jnp.float32),
        pltpu.VMEM((2, _C, _HIDDEN), jnp.float32),
        pltpu.VMEM((_L * _L,), jnp.float32),
        pltpu.VMEM((_EPW,), jnp.float32),
        pltpu.SemaphoreType.DMA,
    ],
)(_edge_body)


def kernel(node_repr, head, rel, tail, sector, entity_type_id,
           rel_emb_W, sector_emb_W, type_emb_W):
    etype2d = entity_type_id.astype(jnp.int32).reshape(_N_NODES, 1)
    aug = pl.pallas_call(
        _augment_body,
        out_shape=jax.ShapeDtypeStruct((_N_NODES, _HIDDEN), jnp.float32),
    )(node_repr, etype2d, type_emb_W)
    relsec = pl.pallas_call(
        _relsec_body,
        out_shape=jax.ShapeDtypeStruct(
            (rel_emb_W.shape[0], _N_SEC, _HIDDEN), jnp.float32),
    )(rel_emb_W, sector_emb_W)
    relsec = relsec.reshape(rel_emb_W.shape[0] * _N_SEC, _HIDDEN)
    return _edge_kernel(
        aug,
        head.astype(jnp.int32), tail.astype(jnp.int32),
        rel.astype(jnp.int32), sector.astype(jnp.int32),
        relsec)


# speedup vs baseline: 30.9815x; 1.0095x over previous
import functools

import jax
import jax.numpy as jnp
from jax import lax
from jax.experimental import pallas as pl
from jax.experimental.pallas import tpu as pltpu
from jax.experimental.pallas import tpu_sc as plsc

_N_NODES = 10000
_N_EDGES = 320000
_HIDDEN = 128
_N_SEC = 50

_NC = 2
_NS = 16
_NW = _NC * _NS
_L = 16

_EPW = _N_EDGES // _NW
_C = 80
_NCHUNK = _EPW // _C


def _augment_body(node_ref, etype_ref, typew_ref, out_ref):
    et = etype_ref[...]
    k = lax.broadcasted_iota(jnp.int32, (et.shape[0], typew_ref.shape[0]), 1)
    onehot = (et == k).astype(jnp.float32)
    out_ref[...] = (node_ref[...] + jnp.dot(
        onehot, typew_ref[...],
        preferred_element_type=jnp.float32)).astype(jnp.bfloat16)


def _relsec_body(relw_ref, secw_ref, out_ref):
    out_ref[...] = (relw_ref[...][:, None, :]
                    + secw_ref[...][None, :, :]).astype(jnp.bfloat16)


def _edge_body(a_hbm, head_hbm, tail_hbm, rel_hbm, sec_hbm, r_hbm,
               out_hbm, head_v, tail_v, rs_v, sec_v, hbuf, tbuf, rbuf,
               partial, out_v, sem):
    wid = lax.axis_index("s") * _NC + lax.axis_index("c")
    base = wid * _EPW
    pltpu.sync_copy(head_hbm.at[pl.ds(base, _EPW)], head_v)
    pltpu.sync_copy(tail_hbm.at[pl.ds(base, _EPW)], tail_v)
    pltpu.sync_copy(rel_hbm.at[pl.ds(base, _EPW)], rs_v)
    pltpu.sync_copy(sec_hbm.at[pl.ds(base, _EPW)], sec_v)

    def rs_body(k, carry):
        sl = pl.ds(k * _L, _L)
        rs_v[sl] = rs_v[sl] * _N_SEC + sec_v[sl]
        return carry

    lax.fori_loop(0, _EPW // _L, rs_body, 0)

    def issue(c, slot):
        off = c * _C
        cps = (
            pltpu.async_copy(a_hbm.at[head_v.at[pl.ds(off, _C)]],
                             hbuf.at[slot], sem),
            pltpu.async_copy(a_hbm.at[tail_v.at[pl.ds(off, _C)]],
                             tbuf.at[slot], sem),
            pltpu.async_copy(r_hbm.at[rs_v.at[pl.ds(off, _C)]],
                             rbuf.at[slot], sem),
        )
        return cps

    def compute(c, slot):
        off = c * _C

        def blk_body(j, carry2):
            def lane_body(l, carry3):
                e = j * _L + l
                acc = jnp.zeros((_L,), jnp.float32)
                for g in range(_HIDDEN // (2 * _L)):
                    sl = pl.ds(g * _L, _L)
                    h0, h1 = plsc.unpack(
                        plsc.bitcast(hbuf[slot, e, sl], jnp.bfloat16),
                        format=plsc.PackFormat.INTERLEAVED)
                    r0, r1 = plsc.unpack(
                        plsc.bitcast(rbuf[slot, e, sl], jnp.bfloat16),
                        format=plsc.PackFormat.INTERLEAVED)
                    t0, t1 = plsc.unpack(
                        plsc.bitcast(tbuf[slot, e, sl], jnp.bfloat16),
                        format=plsc.PackFormat.INTERLEAVED)
                    acc = acc + h0 * r0 * t0
                    acc = acc + h1 * r1 * t1
                partial[pl.ds(l * _L, _L)] = acc
                return carry3

            lax.fori_loop(0, _L, lane_body, 0)
            rowbase = lax.iota(jnp.int32, _L) * _L
            score = jnp.zeros((_L,), jnp.float32)
            for cc in range(_L):
                score = score + plsc.load_gather(partial, [rowbase + cc])
            out_v[pl.ds(off + j * _L, _L)] = score
            return carry2

        lax.fori_loop(0, _C // _L, blk_body, 0)

    def chunk_body(c, carry):
        slot = lax.rem(c, 2)
        cps = issue(c, slot)

        @pl.when(c > 0)
        def _():
            compute(c - 1, 1 - slot)

        for cp in cps:
            cp.wait()
        return carry

    lax.fori_loop(0, _NCHUNK, chunk_body, 0)
    compute(_NCHUNK - 1, lax.rem(_NCHUNK - 1, 2))
    pltpu.sync_copy(out_v, out_hbm.at[pl.ds(base, _EPW)])


_edge_kernel = functools.partial(
    pl.kernel,
    out_type=jax.ShapeDtypeStruct((_N_EDGES,), jnp.float32),
    mesh=plsc.VectorSubcoreMesh(
        core_axis_name="c", subcore_axis_name="s",
        num_cores=_NC, num_subcores=_NS),
    compiler_params=pltpu.CompilerParams(
        needs_layout_passes=False, use_tc_tiling_on_sc=False),
    scratch_types=[
        pltpu.VMEM((_EPW,), jnp.int32),
        pltpu.VMEM((_EPW,), jnp.int32),
        pltpu.VMEM((_EPW,), jnp.int32),
        pltpu.VMEM((_EPW,), jnp.int32),
        pltpu.VMEM((2, _C, _HIDDEN // 2), jnp.int32),
        pltpu.VMEM((2, _C, _HIDDEN // 2), jnp.int32),
        pltpu.VMEM((2, _C, _HIDDEN // 2), jnp.int32),
        pltpu.VMEM((_L * _L,), jnp.float32),
        pltpu.VMEM((_EPW,), jnp.float32),
        pltpu.SemaphoreType.DMA,
    ],
)(_edge_body)


def kernel(node_repr, head, rel, tail, sector, entity_type_id,
           rel_emb_W, sector_emb_W, type_emb_W):
    etype2d = entity_type_id.astype(jnp.int32).reshape(_N_NODES, 1)
    aug = pl.pallas_call(
        _augment_body,
        out_shape=jax.ShapeDtypeStruct((_N_NODES, _HIDDEN), jnp.bfloat16),
    )(node_repr, etype2d, type_emb_W)
    relsec = pl.pallas_call(
        _relsec_body,
        out_shape=jax.ShapeDtypeStruct(
            (rel_emb_W.shape[0], _N_SEC, _HIDDEN), jnp.bfloat16),
    )(rel_emb_W, sector_emb_W)
    relsec = relsec.reshape(rel_emb_W.shape[0] * _N_SEC, _HIDDEN)
    aug = lax.bitcast_convert_type(
        aug.reshape(_N_NODES, _HIDDEN // 2, 2), jnp.int32)
    relsec = lax.bitcast_convert_type(
        relsec.reshape(relsec.shape[0], _HIDDEN // 2, 2), jnp.int32)
    return _edge_kernel(
        aug,
        head.astype(jnp.int32), tail.astype(jnp.int32),
        rel.astype(jnp.int32), sector.astype(jnp.int32),
        relsec)


# speedup vs baseline: 31.8760x; 1.0289x over previous
import functools

import jax
import jax.numpy as jnp
from jax import lax
from jax.experimental import pallas as pl
from jax.experimental.pallas import tpu as pltpu
from jax.experimental.pallas import tpu_sc as plsc

_N_NODES = 10000
_N_EDGES = 320000
_HIDDEN = 128
_N_SEC = 50

_NC = 2
_NS = 16
_NW = _NC * _NS
_L = 16

_EPW = _N_EDGES // _NW
_C = 80
_NCHUNK = _EPW // _C


def _augment_body(node_ref, etype_ref, typew_ref, out_ref):
    et = etype_ref[...]
    k = lax.broadcasted_iota(jnp.int32, (et.shape[0], typew_ref.shape[0]), 1)
    onehot = (et == k).astype(jnp.float32)
    out_ref[...] = (node_ref[...] + jnp.dot(
        onehot, typew_ref[...],
        preferred_element_type=jnp.float32)).astype(jnp.bfloat16)


def _relsec_body(relw_ref, secw_ref, out_ref):
    out_ref[...] = (relw_ref[...][:, None, :]
                    + secw_ref[...][None, :, :]).astype(jnp.bfloat16)


def _edge_body(a_hbm, head_hbm, tail_hbm, rel_hbm, sec_hbm, r_hbm,
               out_hbm, head_v, tail_v, rs_v, sec_v, hbuf, tbuf, rbuf,
               partial, out_v, sem):
    wid = lax.axis_index("s") * _NC + lax.axis_index("c")
    base = wid * _EPW
    pltpu.sync_copy(head_hbm.at[pl.ds(base, _EPW)], head_v)
    pltpu.sync_copy(tail_hbm.at[pl.ds(base, _EPW)], tail_v)
    pltpu.sync_copy(rel_hbm.at[pl.ds(base, _EPW)], rs_v)
    pltpu.sync_copy(sec_hbm.at[pl.ds(base, _EPW)], sec_v)

    def rs_body(k, carry):
        sl = pl.ds(k * _L, _L)
        rs_v[sl] = rs_v[sl] * _N_SEC + sec_v[sl]
        return carry

    lax.fori_loop(0, _EPW // _L, rs_body, 0)

    def issue(c, slot):
        off = c * _C
        cps = (
            pltpu.async_copy(a_hbm.at[head_v.at[pl.ds(off, _C)]],
                             hbuf.at[slot], sem),
            pltpu.async_copy(a_hbm.at[tail_v.at[pl.ds(off, _C)]],
                             tbuf.at[slot], sem),
            pltpu.async_copy(r_hbm.at[rs_v.at[pl.ds(off, _C)]],
                             rbuf.at[slot], sem),
        )
        return cps

    def compute(c, slot):
        off = c * _C

        def blk_body(j, carry2):
            def lane_body(l, carry3):
                e = j * _L + l
                acc = jnp.zeros((_L,), jnp.float32)
                for g in range(_HIDDEN // (2 * _L)):
                    sl = pl.ds(g * _L, _L)
                    hb = plsc.bitcast(hbuf[slot, e, sl], jnp.bfloat16)
                    rb = plsc.bitcast(rbuf[slot, e, sl], jnp.bfloat16)
                    tb = plsc.bitcast(tbuf[slot, e, sl], jnp.bfloat16)
                    p0, p1 = plsc.unpack(
                        hb * rb * tb, format=plsc.PackFormat.INTERLEAVED)
                    acc = acc + p0 + p1
                partial[pl.ds(l * _L, _L)] = acc
                return carry3

            lax.fori_loop(0, _L, lane_body, 0, unroll=4)
            rowbase = lax.iota(jnp.int32, _L) * _L
            score = jnp.zeros((_L,), jnp.float32)
            for cc in range(_L):
                score = score + plsc.load_gather(partial, [rowbase + cc])
            out_v[pl.ds(off + j * _L, _L)] = score
            return carry2

        lax.fori_loop(0, _C // _L, blk_body, 0)

    def chunk_body(c, carry):
        slot = lax.rem(c, 2)
        cps = issue(c, slot)

        @pl.when(c > 0)
        def _():
            compute(c - 1, 1 - slot)

        for cp in cps:
            cp.wait()
        return carry

    lax.fori_loop(0, _NCHUNK, chunk_body, 0)
    compute(_NCHUNK - 1, lax.rem(_NCHUNK - 1, 2))
    pltpu.sync_copy(out_v, out_hbm.at[pl.ds(base, _EPW)])


_edge_kernel = functools.partial(
    pl.kernel,
    out_type=jax.ShapeDtypeStruct((_N_EDGES,), jnp.float32),
    mesh=plsc.VectorSubcoreMesh(
        core_axis_name="c", subcore_axis_name="s",
        num_cores=_NC, num_subcores=_NS),
    compiler_params=pltpu.CompilerParams(
        needs_layout_passes=False, use_tc_tiling_on_sc=False),
    scratch_types=[
        pltpu.VMEM((_EPW,), jnp.int32),
        pltpu.VMEM((_EPW,), jnp.int32),
        pltpu.VMEM((_EPW,), jnp.int32),
        pltpu.VMEM((_EPW,), jnp.int32),
        pltpu.VMEM((2, _C, _HIDDEN // 2), jnp.int32),
        pltpu.VMEM((2, _C, _HIDDEN // 2), jnp.int32),
        pltpu.VMEM((2, _C, _HIDDEN // 2), jnp.int32),
        pltpu.VMEM((_L * _L,), jnp.float32),
        pltpu.VMEM((_EPW,), jnp.float32),
        pltpu.SemaphoreType.DMA,
    ],
)(_edge_body)


def kernel(node_repr, head, rel, tail, sector, entity_type_id,
           rel_emb_W, sector_emb_W, type_emb_W):
    etype2d = entity_type_id.astype(jnp.int32).reshape(_N_NODES, 1)
    aug = pl.pallas_call(
        _augment_body,
        out_shape=jax.ShapeDtypeStruct((_N_NODES, _HIDDEN), jnp.bfloat16),
    )(node_repr, etype2d, type_emb_W)
    relsec = pl.pallas_call(
        _relsec_body,
        out_shape=jax.ShapeDtypeStruct(
            (rel_emb_W.shape[0], _N_SEC, _HIDDEN), jnp.bfloat16),
    )(rel_emb_W, sector_emb_W)
    relsec = relsec.reshape(rel_emb_W.shape[0] * _N_SEC, _HIDDEN)
    aug = lax.bitcast_convert_type(
        aug.reshape(_N_NODES, _HIDDEN // 2, 2), jnp.int32)
    relsec = lax.bitcast_convert_type(
        relsec.reshape(relsec.shape[0], _HIDDEN // 2, 2), jnp.int32)
    return _edge_kernel(
        aug,
        head.astype(jnp.int32), tail.astype(jnp.int32),
        rel.astype(jnp.int32), sector.astype(jnp.int32),
        relsec)
